# baseline (speedup 1.0000x reference)
import functools

import jax
import jax.numpy as jnp
from jax import lax
from jax.experimental import pallas as pl
from jax.experimental.pallas import tpu as pltpu
from jax.experimental.pallas import tpu_sc as plsc

N = 10000
D = 128
E = 320000
NC = 2
NS = 16
NW = NC * NS
CH = 128
RPT = 80
ROWS = RPT * NW
EP = ROWS * CH
NA = 10112
NPT = NA // NS
PAD_DST = N + 8
CW = 16
AGG_SPLIT0 = 48
EPS = 1e-5

_MESH_KW = dict(core_axis_name="c", subcore_axis_name="s", num_cores=NC,
                num_subcores=NS)


@functools.lru_cache(maxsize=None)
def _make_agg(rpt0=RPT, rpt1=RPT):
    NBUF = 2
    rptmax = max(rpt0, rpt1)

    def body(h_hbm, src_hbm, dst_hbm, out_hbm,
             srcv, d0, d1, r0, r1, acc,
             sg0, sg1, sd0, sd1, ss0, ss1):
        c = lax.axis_index("c")
        s = lax.axis_index("s")
        row0 = jnp.where(c == 0, s * rpt0, NS * rpt0 + s * rpt1)
        rpt = jnp.where(c == 0, rpt0, rpt1)
        dsti = (d0, d1)
        rows = (r0, r1)
        semg = (sg0, sg1)
        semd = (sd0, sd1)
        sems = (ss0, ss1)
        zvec = jnp.zeros((16,), jnp.float32)

        def zstore(i, carry):
            r0[i // 8, pl.ds((i % 8) * 16, 16)] = zvec
            return carry

        lax.fori_loop(0, CH * 8, zstore, 0)
        for q in range(NPT // CH):
            pltpu.sync_copy(r0, acc.at[pl.ds(s * NPT + q * CH, CH)])
        _rem = NPT % CH
        if _rem:
            pltpu.sync_copy(r0.at[pl.ds(0, _rem)],
                            acc.at[pl.ds(s * NPT + (NPT // CH) * CH, _rem)])
        pltpu.sync_copy(src_hbm.at[pl.ds(row0 * CH, rptmax * CH)], srcv)
        plsc.subcore_barrier()

        def gfire(k, b):
            pltpu.async_copy(h_hbm.at[srcv.at[pl.ds(k * CH, CH)]],
                             rows[b], semg[b])

        def gwait(k, b):
            pltpu.make_async_copy(h_hbm.at[srcv.at[pl.ds(k * CH, CH)]],
                                  rows[b], semg[b]).wait()

        def dfire(k, b):
            pltpu.async_copy(dst_hbm.at[pl.ds((row0 + k) * CH, CH)],
                             dsti[b], semd[b])

        def dwait(k, b):
            pltpu.make_async_copy(dst_hbm.at[pl.ds((row0 + k) * CH, CH)],
                                  dsti[b], semd[b]).wait()

        for b in range(NBUF):

            @pl.when(b < rpt)
            def _prime():
                dfire(b, b)
                gfire(b, b)

        def outer(j, carry):
            for b in range(NBUF):
                k = NBUF * j + b

                @pl.when(k < rpt)
                def _step():
                    gwait(k, b)
                    dwait(k, b)
                    pltpu.async_copy(rows[b], acc.at[dsti[b]], sems[b],
                                     add=True)

                    @pl.when(k + NBUF < rpt)
                    def _refire():
                        pltpu.make_async_copy(rows[b], acc.at[dsti[b]],
                                              sems[b]).wait()
                        dfire(k + NBUF, b)
                        gfire(k + NBUF, b)

                    @pl.when(k + NBUF >= rpt)
                    def _drain():
                        pltpu.make_async_copy(rows[b], acc.at[dsti[b]],
                                              sems[b]).wait()
            return carry

        lax.fori_loop(0, (rptmax + NBUF - 1) // NBUF, outer, 0)
        plsc.subcore_barrier()
        pltpu.sync_copy(acc.at[pl.ds(s * NPT, NPT)],
                        out_hbm.at[c, pl.ds(s * NPT, NPT)])

    return pl.kernel(
        body,
        out_type=jax.ShapeDtypeStruct((NC, NA, D), jnp.float32),
        mesh=plsc.VectorSubcoreMesh(**_MESH_KW),
        scratch_types=[
            pltpu.VMEM((rptmax * CH,), jnp.int32),
            pltpu.VMEM((CH,), jnp.int32),
            pltpu.VMEM((CH,), jnp.int32),
            pltpu.VMEM((CH, D), jnp.float32),
            pltpu.VMEM((CH, D), jnp.float32),
            pltpu.VMEM_SHARED((NA, D), jnp.float32),
            pltpu.SemaphoreType.DMA,
            pltpu.SemaphoreType.DMA,
            pltpu.SemaphoreType.DMA,
            pltpu.SemaphoreType.DMA,
            pltpu.SemaphoreType.DMA,
            pltpu.SemaphoreType.DMA,
        ],
    )


@functools.lru_cache(maxsize=None)
def _make_cnt():
    def body(dst_hbm, ones_hbm, out_hbm, d0, d1, onesv, acc,
             sd0, sd1, ss0, ss1):
        c = lax.axis_index("c")
        s = lax.axis_index("s")
        w = s * NC + c
        dsti = (d0, d1)
        semd = (sd0, sd1)
        sems = (ss0, ss1)
        zvec = jnp.zeros((16,), jnp.float32)

        def zstore(i, carry):
            onesv[i // 8, pl.ds((i % 8) * 16, 16)] = zvec
            return carry

        lax.fori_loop(0, CH * 8, zstore, 0)
        for q in range(NPT // CH):
            pltpu.sync_copy(onesv, acc.at[pl.ds(s * NPT + q * CH, CH)])
        _rem = NPT % CH
        if _rem:
            pltpu.sync_copy(onesv.at[pl.ds(0, _rem)],
                            acc.at[pl.ds(s * NPT + (NPT // CH) * CH, _rem)])
        pltpu.sync_copy(ones_hbm, onesv)
        plsc.subcore_barrier()

        def dfire(k, b):
            pltpu.async_copy(dst_hbm.at[pl.ds((w * RPT + k) * CH, CH)],
                             dsti[b], semd[b])

        def dwait(k, b):
            pltpu.make_async_copy(dst_hbm.at[pl.ds((w * RPT + k) * CH, CH)],
                                  dsti[b], semd[b]).wait()

        for b in range(2):
            dfire(b, b)

        def outer(j, carry):
            for b in range(2):
                k = 2 * j + b

                @pl.when(k < RPT)
                def _step():
                    dwait(k, b)
                    pltpu.async_copy(onesv, acc.at[dsti[b]], sems[b],
                                     add=True)

                    @pl.when(k + 2 < RPT)
                    def _refire():
                        pltpu.make_async_copy(onesv, acc.at[dsti[b]],
                                              sems[b]).wait()
                        dfire(k + 2, b)

                    @pl.when(k + 2 >= RPT)
                    def _drain():
                        pltpu.make_async_copy(onesv, acc.at[dsti[b]],
                                              sems[b]).wait()
            return carry

        lax.fori_loop(0, (RPT + 1) // 2, outer, 0)
        plsc.subcore_barrier()
        pltpu.sync_copy(acc.at[pl.ds(s * NPT, NPT)],
                        out_hbm.at[c, pl.ds(s * NPT, NPT)])

    return pl.kernel(
        body,
        out_type=jax.ShapeDtypeStruct((NC, NA, D), jnp.float32),
        mesh=plsc.VectorSubcoreMesh(**_MESH_KW),
        scratch_types=[
            pltpu.VMEM((CH,), jnp.int32),
            pltpu.VMEM((CH,), jnp.int32),
            pltpu.VMEM((CH, D), jnp.float32),
            pltpu.VMEM_SHARED((NA, D), jnp.float32),
            pltpu.SemaphoreType.DMA,
            pltpu.SemaphoreType.DMA,
            pltpu.SemaphoreType.DMA,
            pltpu.SemaphoreType.DMA,
        ],
    )


def _dot_t(a, w):
    return lax.dot_general(a, w, (((1,), (1,)), ((), ())),
                           preferred_element_type=jnp.float32,
                           precision=lax.Precision.HIGHEST)


RB = 2000
NB = N // RB


def _make_dense(first, relu):
    def body(agg_ref, c_ref, x_ref, wl_ref, bl_ref, wr_ref, g_ref, be_ref,
             *out_and_scratch):
        if first:
            h_ref, inv_ref, hpre, stats, invs = out_and_scratch
        else:
            h_ref, hpre, stats, invs = out_and_scratch
        i = pl.program_id(0)

        @pl.when(i < NB)
        def _compute():
            if first:
                cnt = (c_ref[0] + c_ref[1])[:, :CW]
                inv = 1.0 / jnp.maximum(cnt, 1.0)
                invs[pl.ds(i * RB, RB)] = inv
            else:
                inv = c_ref[...]
            mean = (agg_ref[0] + agg_ref[1]) * inv[:, 0:1]
            h = (_dot_t(mean, wl_ref[...]) + bl_ref[...]
                 + _dot_t(x_ref[...], wr_ref[...]))
            hpre[pl.ds(i * RB, RB)] = h
            colsum = jnp.sum(h, axis=0, keepdims=True)

            @pl.when(i == 0)
            def _init():
                stats[0:1] = colsum

            @pl.when(i > 0)
            def _accum():
                stats[0:1] += colsum

        @pl.when(i == NB)
        def _finalize():
            mu = stats[0:1] / float(N)
            xc = hpre[...] - mu
            var = jnp.sum(xc * xc, axis=0, keepdims=True) / float(N)
            scale = lax.rsqrt(var + EPS) * g_ref[...]
            out = xc * scale + be_ref[...]
            h_ref[...] = jnp.maximum(out, 0.0) if relu else out
            if first:
                inv_ref[...] = invs[...]

    blk = lambda i: (0, jnp.minimum(i, NB - 1), 0)
    rowblk = lambda i: (jnp.minimum(i, NB - 1), 0)
    full = lambda i: (0, 0)
    in_specs = [
        pl.BlockSpec((NC, RB, D), blk),
        (pl.BlockSpec((NC, RB, D), blk) if first
         else pl.BlockSpec((RB, CW), rowblk)),
        pl.BlockSpec((RB, D), rowblk),
        pl.BlockSpec((D, D), full),
        pl.BlockSpec((1, D), full),
        pl.BlockSpec((D, D), full),
        pl.BlockSpec((1, D), full),
        pl.BlockSpec((1, D), full),
    ]
    out_shape = jax.ShapeDtypeStruct((N, D), jnp.float32)
    out_specs = pl.BlockSpec((N, D), full)
    if first:
        out_shape = (out_shape, jax.ShapeDtypeStruct((N, CW), jnp.float32))
        out_specs = (out_specs, pl.BlockSpec((N, CW), full))
    return pl.pallas_call(
        body,
        grid=(NB + 1,),
        in_specs=in_specs,
        out_specs=out_specs,
        out_shape=out_shape,
        scratch_shapes=[
            pltpu.VMEM((N, D), jnp.float32),
            pltpu.VMEM((8, D), jnp.float32),
            pltpu.VMEM((N, CW), jnp.float32),
        ],
    )


_dense0 = _make_dense(True, True)
_dense_relu = _make_dense(False, True)
_dense_final = _make_dense(False, False)


def kernel(x, edge_index, W0l, b0, W0r, g0, be0, W1l, b1, W1r, g1, be1,
           W2l, b2, W2r, g2, be2):
    ei = edge_index.astype(jnp.int32)
    pad = EP - E
    src = jnp.concatenate([ei[0], jnp.zeros((pad + RPT * CH,), jnp.int32)])
    dst = jnp.concatenate([ei[1], jnp.full((pad,), PAD_DST, jnp.int32)])
    ones_r = jnp.ones((CH, D), jnp.float32)

    agg = _make_agg(AGG_SPLIT0, 2 * RPT - AGG_SPLIT0)
    cnt2 = _make_cnt()(dst, ones_r)
    agg0 = agg(x, src, dst)
    h1, inv = _dense0(agg0, cnt2, x, W0l, b0.reshape(1, D), W0r,
                      g0.reshape(1, D), be0.reshape(1, D))
    agg1 = agg(h1, src, dst)
    h2 = _dense_relu(agg1, inv, h1, W1l, b1.reshape(1, D), W1r,
                     g1.reshape(1, D), be1.reshape(1, D))
    agg2 = agg(h2, src, dst)
    h3 = _dense_final(agg2, inv, h2, W2l, b2.reshape(1, D), W2r,
                      g2.reshape(1, D), be2.reshape(1, D))
    return h3


# speedup vs baseline: 1.0755x; 1.0755x over previous
import functools

import jax
import jax.numpy as jnp
from jax import lax
from jax.experimental import pallas as pl
from jax.experimental.pallas import tpu as pltpu
from jax.experimental.pallas import tpu_sc as plsc

N = 10000
D = 128
E = 320000
NC = 2
NS = 16
NW = NC * NS
CH = 128
RPT = 80
ROWS = RPT * NW
EP = ROWS * CH
NA = 10112
NPT = NA // NS
PAD_DST = N + 8
CW = 16
AGG_SPLIT0 = 112
EPS = 1e-5

_MESH_KW = dict(core_axis_name="c", subcore_axis_name="s", num_cores=NC,
                num_subcores=NS)


@functools.lru_cache(maxsize=None)
def _make_agg(rpt0=RPT, rpt1=RPT):
    NBUF = 2
    rptmax = max(rpt0, rpt1)

    def body(h_hbm, src_hbm, dst_hbm, out_hbm,
             srcv, d0, d1, r0, r1, acc,
             sg0, sg1, sd0, sd1, ss0, ss1):
        c = lax.axis_index("c")
        s = lax.axis_index("s")
        row0 = jnp.where(c == 0, s * rpt0, NS * rpt0 + s * rpt1)
        rpt = jnp.where(c == 0, rpt0, rpt1)
        dsti = (d0, d1)
        rows = (r0, r1)
        semg = (sg0, sg1)
        semd = (sd0, sd1)
        sems = (ss0, ss1)
        zvec = jnp.zeros((16,), jnp.float32)

        def zstore(i, carry):
            r0[i // 8, pl.ds((i % 8) * 16, 16)] = zvec
            return carry

        lax.fori_loop(0, CH * 8, zstore, 0)
        for q in range(NPT // CH):
            pltpu.sync_copy(r0, acc.at[pl.ds(s * NPT + q * CH, CH)])
        _rem = NPT % CH
        if _rem:
            pltpu.sync_copy(r0.at[pl.ds(0, _rem)],
                            acc.at[pl.ds(s * NPT + (NPT // CH) * CH, _rem)])
        pltpu.sync_copy(src_hbm.at[pl.ds(row0 * CH, rptmax * CH)], srcv)
        plsc.subcore_barrier()

        def gfire(k, b):
            pltpu.async_copy(h_hbm.at[srcv.at[pl.ds(k * CH, CH)]],
                             rows[b], semg[b])

        def gwait(k, b):
            pltpu.make_async_copy(h_hbm.at[srcv.at[pl.ds(k * CH, CH)]],
                                  rows[b], semg[b]).wait()

        def dfire(k, b):
            pltpu.async_copy(dst_hbm.at[pl.ds((row0 + k) * CH, CH)],
                             dsti[b], semd[b])

        def dwait(k, b):
            pltpu.make_async_copy(dst_hbm.at[pl.ds((row0 + k) * CH, CH)],
                                  dsti[b], semd[b]).wait()

        for b in range(NBUF):

            @pl.when(b < rpt)
            def _prime():
                dfire(b, b)
                gfire(b, b)

        def outer(j, carry):
            for b in range(NBUF):
                k = NBUF * j + b

                @pl.when(k < rpt)
                def _step():
                    gwait(k, b)
                    dwait(k, b)
                    pltpu.async_copy(rows[b], acc.at[dsti[b]], sems[b],
                                     add=True)

                    @pl.when(k + NBUF < rpt)
                    def _refire():
                        pltpu.make_async_copy(rows[b], acc.at[dsti[b]],
                                              sems[b]).wait()
                        dfire(k + NBUF, b)
                        gfire(k + NBUF, b)

                    @pl.when(k + NBUF >= rpt)
                    def _drain():
                        pltpu.make_async_copy(rows[b], acc.at[dsti[b]],
                                              sems[b]).wait()
            return carry

        lax.fori_loop(0, (rptmax + NBUF - 1) // NBUF, outer, 0)
        plsc.subcore_barrier()
        pltpu.sync_copy(acc.at[pl.ds(s * NPT, NPT)],
                        out_hbm.at[c, pl.ds(s * NPT, NPT)])

    return pl.kernel(
        body,
        out_type=jax.ShapeDtypeStruct((NC, NA, D), jnp.float32),
        mesh=plsc.VectorSubcoreMesh(**_MESH_KW),
        scratch_types=[
            pltpu.VMEM((rptmax * CH,), jnp.int32),
            pltpu.VMEM((CH,), jnp.int32),
            pltpu.VMEM((CH,), jnp.int32),
            pltpu.VMEM((CH, D), jnp.float32),
            pltpu.VMEM((CH, D), jnp.float32),
            pltpu.VMEM_SHARED((NA, D), jnp.float32),
            pltpu.SemaphoreType.DMA,
            pltpu.SemaphoreType.DMA,
            pltpu.SemaphoreType.DMA,
            pltpu.SemaphoreType.DMA,
            pltpu.SemaphoreType.DMA,
            pltpu.SemaphoreType.DMA,
        ],
    )


@functools.lru_cache(maxsize=None)
def _make_cnt():
    def body(dst_hbm, ones_hbm, out_hbm, d0, d1, onesv, acc,
             sd0, sd1, ss0, ss1):
        c = lax.axis_index("c")
        s = lax.axis_index("s")
        w = s * NC + c
        dsti = (d0, d1)
        semd = (sd0, sd1)
        sems = (ss0, ss1)
        zvec = jnp.zeros((16,), jnp.float32)

        def zstore(i, carry):
            onesv[i // 8, pl.ds((i % 8) * 16, 16)] = zvec
            return carry

        lax.fori_loop(0, CH * 8, zstore, 0)
        for q in range(NPT // CH):
            pltpu.sync_copy(onesv, acc.at[pl.ds(s * NPT + q * CH, CH)])
        _rem = NPT % CH
        if _rem:
            pltpu.sync_copy(onesv.at[pl.ds(0, _rem)],
                            acc.at[pl.ds(s * NPT + (NPT // CH) * CH, _rem)])
        pltpu.sync_copy(ones_hbm, onesv)
        plsc.subcore_barrier()

        def dfire(k, b):
            pltpu.async_copy(dst_hbm.at[pl.ds((w * RPT + k) * CH, CH)],
                             dsti[b], semd[b])

        def dwait(k, b):
            pltpu.make_async_copy(dst_hbm.at[pl.ds((w * RPT + k) * CH, CH)],
                                  dsti[b], semd[b]).wait()

        for b in range(2):
            dfire(b, b)

        def outer(j, carry):
            for b in range(2):
                k = 2 * j + b

                @pl.when(k < RPT)
                def _step():
                    dwait(k, b)
                    pltpu.async_copy(onesv, acc.at[dsti[b]], sems[b],
                                     add=True)

                    @pl.when(k + 2 < RPT)
                    def _refire():
                        pltpu.make_async_copy(onesv, acc.at[dsti[b]],
                                              sems[b]).wait()
                        dfire(k + 2, b)

                    @pl.when(k + 2 >= RPT)
                    def _drain():
                        pltpu.make_async_copy(onesv, acc.at[dsti[b]],
                                              sems[b]).wait()
            return carry

        lax.fori_loop(0, (RPT + 1) // 2, outer, 0)
        plsc.subcore_barrier()
        pltpu.sync_copy(acc.at[pl.ds(s * NPT, NPT)],
                        out_hbm.at[c, pl.ds(s * NPT, NPT)])

    return pl.kernel(
        body,
        out_type=jax.ShapeDtypeStruct((NC, NA, D), jnp.float32),
        mesh=plsc.VectorSubcoreMesh(**_MESH_KW),
        scratch_types=[
            pltpu.VMEM((CH,), jnp.int32),
            pltpu.VMEM((CH,), jnp.int32),
            pltpu.VMEM((CH, D), jnp.float32),
            pltpu.VMEM_SHARED((NA, D), jnp.float32),
            pltpu.SemaphoreType.DMA,
            pltpu.SemaphoreType.DMA,
            pltpu.SemaphoreType.DMA,
            pltpu.SemaphoreType.DMA,
        ],
    )


def _dot_t(a, w):
    return lax.dot_general(a, w, (((1,), (1,)), ((), ())),
                           preferred_element_type=jnp.float32,
                           precision=lax.Precision.HIGHEST)


RB = 2000
NB = N // RB


def _make_dense(first, relu):
    def body(agg_ref, c_ref, x_ref, wl_ref, bl_ref, wr_ref, g_ref, be_ref,
             *out_and_scratch):
        if first:
            h_ref, inv_ref, hpre, stats, invs = out_and_scratch
        else:
            h_ref, hpre, stats, invs = out_and_scratch
        i = pl.program_id(0)

        @pl.when(i < NB)
        def _compute():
            if first:
                cnt = (c_ref[0] + c_ref[1])[:, :CW]
                inv = 1.0 / jnp.maximum(cnt, 1.0)
                invs[pl.ds(i * RB, RB)] = inv
            else:
                inv = c_ref[...]
            mean = (agg_ref[0] + agg_ref[1]) * inv[:, 0:1]
            h = (_dot_t(mean, wl_ref[...]) + bl_ref[...]
                 + _dot_t(x_ref[...], wr_ref[...]))
            hpre[pl.ds(i * RB, RB)] = h
            colsum = jnp.sum(h, axis=0, keepdims=True)

            @pl.when(i == 0)
            def _init():
                stats[0:1] = colsum

            @pl.when(i > 0)
            def _accum():
                stats[0:1] += colsum

        @pl.when(i == NB)
        def _finalize():
            mu = stats[0:1] / float(N)
            xc = hpre[...] - mu
            var = jnp.sum(xc * xc, axis=0, keepdims=True) / float(N)
            scale = lax.rsqrt(var + EPS) * g_ref[...]
            out = xc * scale + be_ref[...]
            h_ref[...] = jnp.maximum(out, 0.0) if relu else out
            if first:
                inv_ref[...] = invs[...]

    blk = lambda i: (0, jnp.minimum(i, NB - 1), 0)
    rowblk = lambda i: (jnp.minimum(i, NB - 1), 0)
    full = lambda i: (0, 0)
    in_specs = [
        pl.BlockSpec((NC, RB, D), blk),
        (pl.BlockSpec((NC, RB, D), blk) if first
         else pl.BlockSpec((RB, CW), rowblk)),
        pl.BlockSpec((RB, D), rowblk),
        pl.BlockSpec((D, D), full),
        pl.BlockSpec((1, D), full),
        pl.BlockSpec((D, D), full),
        pl.BlockSpec((1, D), full),
        pl.BlockSpec((1, D), full),
    ]
    out_shape = jax.ShapeDtypeStruct((N, D), jnp.float32)
    out_specs = pl.BlockSpec((N, D), full)
    if first:
        out_shape = (out_shape, jax.ShapeDtypeStruct((N, CW), jnp.float32))
        out_specs = (out_specs, pl.BlockSpec((N, CW), full))
    return pl.pallas_call(
        body,
        grid=(NB + 1,),
        in_specs=in_specs,
        out_specs=out_specs,
        out_shape=out_shape,
        scratch_shapes=[
            pltpu.VMEM((N, D), jnp.float32),
            pltpu.VMEM((8, D), jnp.float32),
            pltpu.VMEM((N, CW), jnp.float32),
        ],
    )


_dense0 = _make_dense(True, True)
_dense_relu = _make_dense(False, True)
_dense_final = _make_dense(False, False)


def kernel(x, edge_index, W0l, b0, W0r, g0, be0, W1l, b1, W1r, g1, be1,
           W2l, b2, W2r, g2, be2):
    ei = edge_index.astype(jnp.int32)
    pad = EP - E
    src = jnp.concatenate([ei[0], jnp.zeros((pad + RPT * CH,), jnp.int32)])
    dst = jnp.concatenate([ei[1], jnp.full((pad,), PAD_DST, jnp.int32)])
    ones_r = jnp.ones((CH, D), jnp.float32)

    agg = _make_agg(AGG_SPLIT0, 2 * RPT - AGG_SPLIT0)
    cnt2 = _make_cnt()(dst, ones_r)
    agg0 = agg(x, src, dst)
    h1, inv = _dense0(agg0, cnt2, x, W0l, b0.reshape(1, D), W0r,
                      g0.reshape(1, D), be0.reshape(1, D))
    agg1 = agg(h1, src, dst)
    h2 = _dense_relu(agg1, inv, h1, W1l, b1.reshape(1, D), W1r,
                     g1.reshape(1, D), be1.reshape(1, D))
    agg2 = agg(h2, src, dst)
    h3 = _dense_final(agg2, inv, h2, W2l, b2.reshape(1, D), W2r,
                      g2.reshape(1, D), be2.reshape(1, D))
    return h3


# speedup vs baseline: 1.0879x; 1.0115x over previous
import functools

import jax
import jax.numpy as jnp
from jax import lax
from jax.experimental import pallas as pl
from jax.experimental.pallas import tpu as pltpu
from jax.experimental.pallas import tpu_sc as plsc

N = 10000
D = 128
E = 320000
NC = 2
NS = 16
NW = NC * NS
CH = 128
RPT = 80
ROWS = RPT * NW
EP = ROWS * CH
NA = 10112
NPT = NA // NS
PAD_DST = N + 8
CW = 16
AGG_SPLIT0 = 128
EPS = 1e-5

_MESH_KW = dict(core_axis_name="c", subcore_axis_name="s", num_cores=NC,
                num_subcores=NS)


@functools.lru_cache(maxsize=None)
def _make_agg(rpt0=RPT, rpt1=RPT):
    NBUF = 2
    rptmax = max(rpt0, rpt1)

    def body(h_hbm, src_hbm, dst_hbm, out_hbm,
             srcv, d0, d1, r0, r1, acc,
             sg0, sg1, sd0, sd1, ss0, ss1):
        c = lax.axis_index("c")
        s = lax.axis_index("s")
        row0 = jnp.where(c == 0, s * rpt0, NS * rpt0 + s * rpt1)
        rpt = jnp.where(c == 0, rpt0, rpt1)
        dsti = (d0, d1)
        rows = (r0, r1)
        semg = (sg0, sg1)
        semd = (sd0, sd1)
        sems = (ss0, ss1)
        zvec = jnp.zeros((16,), jnp.float32)

        def zstore(i, carry):
            r0[i // 8, pl.ds((i % 8) * 16, 16)] = zvec
            return carry

        lax.fori_loop(0, CH * 8, zstore, 0)
        for q in range(NPT // CH):
            pltpu.sync_copy(r0, acc.at[pl.ds(s * NPT + q * CH, CH)])
        _rem = NPT % CH
        if _rem:
            pltpu.sync_copy(r0.at[pl.ds(0, _rem)],
                            acc.at[pl.ds(s * NPT + (NPT // CH) * CH, _rem)])
        pltpu.sync_copy(src_hbm.at[pl.ds(row0 * CH, rptmax * CH)], srcv)
        plsc.subcore_barrier()

        def gfire(k, b):
            pltpu.async_copy(h_hbm.at[srcv.at[pl.ds(k * CH, CH)]],
                             rows[b], semg[b])

        def gwait(k, b):
            pltpu.make_async_copy(h_hbm.at[srcv.at[pl.ds(k * CH, CH)]],
                                  rows[b], semg[b]).wait()

        def dfire(k, b):
            pltpu.async_copy(dst_hbm.at[pl.ds((row0 + k) * CH, CH)],
                             dsti[b], semd[b])

        def dwait(k, b):
            pltpu.make_async_copy(dst_hbm.at[pl.ds((row0 + k) * CH, CH)],
                                  dsti[b], semd[b]).wait()

        for b in range(NBUF):

            @pl.when(b < rpt)
            def _prime():
                dfire(b, b)
                gfire(b, b)

        def outer(j, carry):
            for b in range(NBUF):
                k = NBUF * j + b

                @pl.when(k < rpt)
                def _step():
                    gwait(k, b)
                    dwait(k, b)
                    pltpu.async_copy(rows[b], acc.at[dsti[b]], sems[b],
                                     add=True)

                    @pl.when(k + NBUF < rpt)
                    def _refire():
                        pltpu.make_async_copy(rows[b], acc.at[dsti[b]],
                                              sems[b]).wait()
                        dfire(k + NBUF, b)
                        gfire(k + NBUF, b)

                    @pl.when(k + NBUF >= rpt)
                    def _drain():
                        pltpu.make_async_copy(rows[b], acc.at[dsti[b]],
                                              sems[b]).wait()
            return carry

        lax.fori_loop(0, (rptmax + NBUF - 1) // NBUF, outer, 0)
        plsc.subcore_barrier()
        pltpu.sync_copy(acc.at[pl.ds(s * NPT, NPT)],
                        out_hbm.at[c, pl.ds(s * NPT, NPT)])

    return pl.kernel(
        body,
        out_type=jax.ShapeDtypeStruct((NC, NA, D), jnp.float32),
        mesh=plsc.VectorSubcoreMesh(**_MESH_KW),
        scratch_types=[
            pltpu.VMEM((rptmax * CH,), jnp.int32),
            pltpu.VMEM((CH,), jnp.int32),
            pltpu.VMEM((CH,), jnp.int32),
            pltpu.VMEM((CH, D), jnp.float32),
            pltpu.VMEM((CH, D), jnp.float32),
            pltpu.VMEM_SHARED((NA, D), jnp.float32),
            pltpu.SemaphoreType.DMA,
            pltpu.SemaphoreType.DMA,
            pltpu.SemaphoreType.DMA,
            pltpu.SemaphoreType.DMA,
            pltpu.SemaphoreType.DMA,
            pltpu.SemaphoreType.DMA,
        ],
    )


@functools.lru_cache(maxsize=None)
def _make_cnt():
    def body(dst_hbm, ones_hbm, out_hbm, d0, d1, onesv, acc,
             sd0, sd1, ss0, ss1):
        c = lax.axis_index("c")
        s = lax.axis_index("s")
        w = s * NC + c
        dsti = (d0, d1)
        semd = (sd0, sd1)
        sems = (ss0, ss1)
        zvec = jnp.zeros((16,), jnp.float32)

        def zstore(i, carry):
            onesv[i // 8, pl.ds((i % 8) * 16, 16)] = zvec
            return carry

        lax.fori_loop(0, CH * 8, zstore, 0)
        for q in range(NPT // CH):
            pltpu.sync_copy(onesv, acc.at[pl.ds(s * NPT + q * CH, CH)])
        _rem = NPT % CH
        if _rem:
            pltpu.sync_copy(onesv.at[pl.ds(0, _rem)],
                            acc.at[pl.ds(s * NPT + (NPT // CH) * CH, _rem)])
        pltpu.sync_copy(ones_hbm, onesv)
        plsc.subcore_barrier()

        def dfire(k, b):
            pltpu.async_copy(dst_hbm.at[pl.ds((w * RPT + k) * CH, CH)],
                             dsti[b], semd[b])

        def dwait(k, b):
            pltpu.make_async_copy(dst_hbm.at[pl.ds((w * RPT + k) * CH, CH)],
                                  dsti[b], semd[b]).wait()

        for b in range(2):
            dfire(b, b)

        def outer(j, carry):
            for b in range(2):
                k = 2 * j + b

                @pl.when(k < RPT)
                def _step():
                    dwait(k, b)
                    pltpu.async_copy(onesv, acc.at[dsti[b]], sems[b],
                                     add=True)

                    @pl.when(k + 2 < RPT)
                    def _refire():
                        pltpu.make_async_copy(onesv, acc.at[dsti[b]],
                                              sems[b]).wait()
                        dfire(k + 2, b)

                    @pl.when(k + 2 >= RPT)
                    def _drain():
                        pltpu.make_async_copy(onesv, acc.at[dsti[b]],
                                              sems[b]).wait()
            return carry

        lax.fori_loop(0, (RPT + 1) // 2, outer, 0)
        plsc.subcore_barrier()
        pltpu.sync_copy(acc.at[pl.ds(s * NPT, NPT)],
                        out_hbm.at[c, pl.ds(s * NPT, NPT)])

    return pl.kernel(
        body,
        out_type=jax.ShapeDtypeStruct((NC, NA, D), jnp.float32),
        mesh=plsc.VectorSubcoreMesh(**_MESH_KW),
        scratch_types=[
            pltpu.VMEM((CH,), jnp.int32),
            pltpu.VMEM((CH,), jnp.int32),
            pltpu.VMEM((CH, D), jnp.float32),
            pltpu.VMEM_SHARED((NA, D), jnp.float32),
            pltpu.SemaphoreType.DMA,
            pltpu.SemaphoreType.DMA,
            pltpu.SemaphoreType.DMA,
            pltpu.SemaphoreType.DMA,
        ],
    )


def _dot_t(a, w):
    return lax.dot_general(a, w, (((1,), (1,)), ((), ())),
                           preferred_element_type=jnp.float32,
                           precision=lax.Precision.HIGHEST)


RB = 2000
NB = N // RB


def _make_dense(first, relu):
    def body(agg_ref, c_ref, x_ref, wl_ref, bl_ref, wr_ref, g_ref, be_ref,
             *out_and_scratch):
        if first:
            h_ref, inv_ref, hpre, stats, invs = out_and_scratch
        else:
            h_ref, hpre, stats, invs = out_and_scratch
        i = pl.program_id(0)

        @pl.when(i < NB)
        def _compute():
            if first:
                cnt = (c_ref[0] + c_ref[1])[:, :CW]
                inv = 1.0 / jnp.maximum(cnt, 1.0)
                invs[pl.ds(i * RB, RB)] = inv
            else:
                inv = c_ref[...]
            mean = (agg_ref[0] + agg_ref[1]) * inv[:, 0:1]
            h = (_dot_t(mean, wl_ref[...]) + bl_ref[...]
                 + _dot_t(x_ref[...], wr_ref[...]))
            hpre[pl.ds(i * RB, RB)] = h
            colsum = jnp.sum(h, axis=0, keepdims=True)

            @pl.when(i == 0)
            def _init():
                stats[0:1] = colsum

            @pl.when(i > 0)
            def _accum():
                stats[0:1] += colsum

        @pl.when(i == NB)
        def _finalize():
            mu = stats[0:1] / float(N)
            xc = hpre[...] - mu
            var = jnp.sum(xc * xc, axis=0, keepdims=True) / float(N)
            scale = lax.rsqrt(var + EPS) * g_ref[...]
            out = xc * scale + be_ref[...]
            h_ref[...] = jnp.maximum(out, 0.0) if relu else out
            if first:
                inv_ref[...] = invs[...]

    blk = lambda i: (0, jnp.minimum(i, NB - 1), 0)
    rowblk = lambda i: (jnp.minimum(i, NB - 1), 0)
    full = lambda i: (0, 0)
    in_specs = [
        pl.BlockSpec((NC, RB, D), blk),
        (pl.BlockSpec((NC, RB, D), blk) if first
         else pl.BlockSpec((RB, CW), rowblk)),
        pl.BlockSpec((RB, D), rowblk),
        pl.BlockSpec((D, D), full),
        pl.BlockSpec((1, D), full),
        pl.BlockSpec((D, D), full),
        pl.BlockSpec((1, D), full),
        pl.BlockSpec((1, D), full),
    ]
    out_shape = jax.ShapeDtypeStruct((N, D), jnp.float32)
    out_specs = pl.BlockSpec((N, D), full)
    if first:
        out_shape = (out_shape, jax.ShapeDtypeStruct((N, CW), jnp.float32))
        out_specs = (out_specs, pl.BlockSpec((N, CW), full))
    return pl.pallas_call(
        body,
        grid=(NB + 1,),
        in_specs=in_specs,
        out_specs=out_specs,
        out_shape=out_shape,
        scratch_shapes=[
            pltpu.VMEM((N, D), jnp.float32),
            pltpu.VMEM((8, D), jnp.float32),
            pltpu.VMEM((N, CW), jnp.float32),
        ],
    )


_dense0 = _make_dense(True, True)
_dense_relu = _make_dense(False, True)
_dense_final = _make_dense(False, False)


def kernel(x, edge_index, W0l, b0, W0r, g0, be0, W1l, b1, W1r, g1, be1,
           W2l, b2, W2r, g2, be2):
    ei = edge_index.astype(jnp.int32)
    pad = EP - E
    src = jnp.concatenate([ei[0], jnp.zeros((pad + RPT * CH,), jnp.int32)])
    dst = jnp.concatenate([ei[1], jnp.full((pad,), PAD_DST, jnp.int32)])
    ones_r = jnp.ones((CH, D), jnp.float32)

    agg = _make_agg(AGG_SPLIT0, 2 * RPT - AGG_SPLIT0)
    cnt2 = _make_cnt()(dst, ones_r)
    agg0 = agg(x, src, dst)
    h1, inv = _dense0(agg0, cnt2, x, W0l, b0.reshape(1, D), W0r,
                      g0.reshape(1, D), be0.reshape(1, D))
    agg1 = agg(h1, src, dst)
    h2 = _dense_relu(agg1, inv, h1, W1l, b1.reshape(1, D), W1r,
                     g1.reshape(1, D), be1.reshape(1, D))
    agg2 = agg(h2, src, dst)
    h3 = _dense_final(agg2, inv, h2, W2l, b2.reshape(1, D), W2r,
                      g2.reshape(1, D), be2.reshape(1, D))
    return h3


# speedup vs baseline: 1.1610x; 1.0672x over previous
import functools

import jax
import jax.numpy as jnp
from jax import lax
from jax.experimental import pallas as pl
from jax.experimental.pallas import tpu as pltpu
from jax.experimental.pallas import tpu_sc as plsc

N = 10000
D = 128
E = 320000
NC = 2
NS = 16
NW = NC * NS
CH = 128
RPT = 80
ROWS = RPT * NW
EP = ROWS * CH
NA = 10112
NPT = NA // NS
PAD_DST = N + 8
CW = 16
AGG_SPLIT0 = 128
EPS = 1e-5

_MESH_KW = dict(core_axis_name="c", subcore_axis_name="s", num_cores=NC,
                num_subcores=NS)


@functools.lru_cache(maxsize=None)
def _make_agg(rpt0=RPT, rpt1=RPT):
    NBUF = 2
    rptmax = max(rpt0, rpt1)

    def body(h_hbm, src_hbm, dst_hbm, out_hbm,
             srcv, d0, d1, r0, r1, acc,
             sg0, sg1, sd0, sd1, ss0, ss1):
        c = lax.axis_index("c")
        s = lax.axis_index("s")
        row0 = jnp.where(c == 0, s * rpt0, NS * rpt0 + s * rpt1)
        rpt = jnp.where(c == 0, rpt0, rpt1)
        dsti = (d0, d1)
        rows = (r0, r1)
        semg = (sg0, sg1)
        semd = (sd0, sd1)
        sems = (ss0, ss1)
        zvec = jnp.zeros((16,), jnp.float32)

        def zstore(i, carry):
            r0[i // 8, pl.ds((i % 8) * 16, 16)] = zvec
            return carry

        lax.fori_loop(0, CH * 8, zstore, 0)
        for q in range(NPT // CH):
            pltpu.sync_copy(r0, acc.at[pl.ds(s * NPT + q * CH, CH)])
        _rem = NPT % CH
        if _rem:
            pltpu.sync_copy(r0.at[pl.ds(0, _rem)],
                            acc.at[pl.ds(s * NPT + (NPT // CH) * CH, _rem)])
        pltpu.sync_copy(src_hbm.at[pl.ds(row0 * CH, rptmax * CH)], srcv)
        plsc.subcore_barrier()

        def gfire(k, b):
            pltpu.async_copy(h_hbm.at[srcv.at[pl.ds(k * CH, CH)]],
                             rows[b], semg[b])

        def gwait(k, b):
            pltpu.make_async_copy(h_hbm.at[srcv.at[pl.ds(k * CH, CH)]],
                                  rows[b], semg[b]).wait()

        def dfire(k, b):
            pltpu.async_copy(dst_hbm.at[pl.ds((row0 + k) * CH, CH)],
                             dsti[b], semd[b])

        def dwait(k, b):
            pltpu.make_async_copy(dst_hbm.at[pl.ds((row0 + k) * CH, CH)],
                                  dsti[b], semd[b]).wait()

        for b in range(NBUF):

            @pl.when(b < rpt)
            def _prime():
                dfire(b, b)
                gfire(b, b)

        def outer(j, carry):
            for b in range(NBUF):
                k = NBUF * j + b

                @pl.when(k < rpt)
                def _step():
                    gwait(k, b)
                    dwait(k, b)
                    pltpu.async_copy(rows[b], acc.at[dsti[b]], sems[b],
                                     add=True)

                    @pl.when(k + NBUF < rpt)
                    def _refire():
                        pltpu.make_async_copy(rows[b], acc.at[dsti[b]],
                                              sems[b]).wait()
                        dfire(k + NBUF, b)
                        gfire(k + NBUF, b)

                    @pl.when(k + NBUF >= rpt)
                    def _drain():
                        pltpu.make_async_copy(rows[b], acc.at[dsti[b]],
                                              sems[b]).wait()
            return carry

        lax.fori_loop(0, (rptmax + NBUF - 1) // NBUF, outer, 0)
        plsc.subcore_barrier()
        pltpu.sync_copy(acc.at[pl.ds(s * NPT, NPT)],
                        out_hbm.at[c, pl.ds(s * NPT, NPT)])

    return pl.kernel(
        body,
        out_type=jax.ShapeDtypeStruct((NC, NA, D), jnp.float32),
        mesh=plsc.VectorSubcoreMesh(**_MESH_KW),
        scratch_types=[
            pltpu.VMEM((rptmax * CH,), jnp.int32),
            pltpu.VMEM((CH,), jnp.int32),
            pltpu.VMEM((CH,), jnp.int32),
            pltpu.VMEM((CH, D), jnp.float32),
            pltpu.VMEM((CH, D), jnp.float32),
            pltpu.VMEM_SHARED((NA, D), jnp.float32),
            pltpu.SemaphoreType.DMA,
            pltpu.SemaphoreType.DMA,
            pltpu.SemaphoreType.DMA,
            pltpu.SemaphoreType.DMA,
            pltpu.SemaphoreType.DMA,
            pltpu.SemaphoreType.DMA,
        ],
    )


@functools.lru_cache(maxsize=None)
def _make_cnt():
    def body(dst_hbm, ones_hbm, out_hbm, d0, d1, onesv, acc,
             sd0, sd1, ss0, ss1):
        c = lax.axis_index("c")
        s = lax.axis_index("s")
        w = s * NC + c
        dsti = (d0, d1)
        semd = (sd0, sd1)
        sems = (ss0, ss1)
        zvec = jnp.zeros((16,), jnp.float32)

        def zstore(i, carry):
            onesv[i // 8, pl.ds((i % 8) * 16, 16)] = zvec
            return carry

        lax.fori_loop(0, CH * 8, zstore, 0)
        for q in range(NPT // CH):
            pltpu.sync_copy(onesv, acc.at[pl.ds(s * NPT + q * CH, CH)])
        _rem = NPT % CH
        if _rem:
            pltpu.sync_copy(onesv.at[pl.ds(0, _rem)],
                            acc.at[pl.ds(s * NPT + (NPT // CH) * CH, _rem)])
        pltpu.sync_copy(ones_hbm, onesv)
        plsc.subcore_barrier()

        def dfire(k, b):
            pltpu.async_copy(dst_hbm.at[pl.ds((w * RPT + k) * CH, CH)],
                             dsti[b], semd[b])

        def dwait(k, b):
            pltpu.make_async_copy(dst_hbm.at[pl.ds((w * RPT + k) * CH, CH)],
                                  dsti[b], semd[b]).wait()

        for b in range(2):
            dfire(b, b)

        def outer(j, carry):
            for b in range(2):
                k = 2 * j + b

                @pl.when(k < RPT)
                def _step():
                    dwait(k, b)
                    pltpu.async_copy(onesv, acc.at[dsti[b]], sems[b],
                                     add=True)

                    @pl.when(k + 2 < RPT)
                    def _refire():
                        pltpu.make_async_copy(onesv, acc.at[dsti[b]],
                                              sems[b]).wait()
                        dfire(k + 2, b)

                    @pl.when(k + 2 >= RPT)
                    def _drain():
                        pltpu.make_async_copy(onesv, acc.at[dsti[b]],
                                              sems[b]).wait()
            return carry

        lax.fori_loop(0, (RPT + 1) // 2, outer, 0)
        plsc.subcore_barrier()
        pltpu.sync_copy(acc.at[pl.ds(s * NPT, NPT)],
                        out_hbm.at[c, pl.ds(s * NPT, NPT)])

    return pl.kernel(
        body,
        out_type=jax.ShapeDtypeStruct((NC, NA, D), jnp.float32),
        mesh=plsc.VectorSubcoreMesh(**_MESH_KW),
        scratch_types=[
            pltpu.VMEM((CH,), jnp.int32),
            pltpu.VMEM((CH,), jnp.int32),
            pltpu.VMEM((CH, D), jnp.float32),
            pltpu.VMEM_SHARED((NA, D), jnp.float32),
            pltpu.SemaphoreType.DMA,
            pltpu.SemaphoreType.DMA,
            pltpu.SemaphoreType.DMA,
            pltpu.SemaphoreType.DMA,
        ],
    )


def _dot_t(a, w):
    return lax.dot_general(a, w, (((1,), (1,)), ((), ())),
                           preferred_element_type=jnp.float32,
                           precision=lax.Precision.HIGHEST)


RB = 2000
NB = N // RB


def _make_dense(first, relu):
    def body(agg_ref, c_ref, x_ref, wl_ref, bl_ref, wr_ref, g_ref, be_ref,
             *out_and_scratch):
        if first:
            h_ref, inv_ref, hpre, stats, invs = out_and_scratch
        else:
            h_ref, hpre, stats, invs = out_and_scratch
        i = pl.program_id(0)

        @pl.when(i < NB)
        def _compute():
            if first:
                cnt = (c_ref[0] + c_ref[1])[:, :CW]
                inv = 1.0 / jnp.maximum(cnt, 1.0)
                invs[pl.ds(i * RB, RB)] = inv
            else:
                inv = c_ref[...]
            mean = (agg_ref[0] + agg_ref[1]) * inv[:, 0:1]
            h = (_dot_t(mean, wl_ref[...]) + bl_ref[...]
                 + _dot_t(x_ref[...], wr_ref[...]))
            hpre[pl.ds(i * RB, RB)] = h
            colsum = jnp.sum(h, axis=0, keepdims=True)

            @pl.when(i == 0)
            def _init():
                stats[0:1] = colsum

            @pl.when(i > 0)
            def _accum():
                stats[0:1] += colsum

        @pl.when(i == NB)
        def _finalize():
            mu = stats[0:1] / float(N)
            xc = hpre[...] - mu
            var = jnp.sum(xc * xc, axis=0, keepdims=True) / float(N)
            scale = lax.rsqrt(var + EPS) * g_ref[...]
            out = xc * scale + be_ref[...]
            h_ref[...] = jnp.maximum(out, 0.0) if relu else out
            if first:
                inv_ref[...] = invs[...]

    blk = lambda i: (0, jnp.minimum(i, NB - 1), 0)
    rowblk = lambda i: (jnp.minimum(i, NB - 1), 0)
    full = lambda i: (0, 0)
    in_specs = [
        pl.BlockSpec((NC, RB, D), blk),
        (pl.BlockSpec((NC, RB, D), blk) if first
         else pl.BlockSpec((RB, CW), rowblk)),
        pl.BlockSpec((RB, D), rowblk),
        pl.BlockSpec((D, D), full),
        pl.BlockSpec((1, D), full),
        pl.BlockSpec((D, D), full),
        pl.BlockSpec((1, D), full),
        pl.BlockSpec((1, D), full),
    ]
    out_shape = jax.ShapeDtypeStruct((N, D), jnp.float32)
    out_specs = pl.BlockSpec((N, D), full)
    if first:
        out_shape = (out_shape, jax.ShapeDtypeStruct((N, CW), jnp.float32))
        out_specs = (out_specs, pl.BlockSpec((N, CW), full))
    return pl.pallas_call(
        body,
        grid=(NB + 1,),
        in_specs=in_specs,
        out_specs=out_specs,
        out_shape=out_shape,
        scratch_shapes=[
            pltpu.VMEM((N, D), jnp.float32),
            pltpu.VMEM((8, D), jnp.float32),
            pltpu.VMEM((N, CW), jnp.float32),
        ],
    )


_dense0 = _make_dense(True, True)
_dense_relu = _make_dense(False, True)
_dense_final = _make_dense(False, False)


def kernel(x, edge_index, W0l, b0, W0r, g0, be0, W1l, b1, W1r, g1, be1,
           W2l, b2, W2r, g2, be2):
    ei = edge_index.astype(jnp.int32)
    pad = EP - E
    src = jnp.concatenate([ei[0], jnp.zeros((pad + 2 * RPT * CH,), jnp.int32)])
    dst = jnp.concatenate([ei[1], jnp.full((pad,), PAD_DST, jnp.int32)])
    ones_r = jnp.ones((CH, D), jnp.float32)

    agg = _make_agg(AGG_SPLIT0, 2 * RPT - AGG_SPLIT0)
    cnt2 = _make_cnt()(dst, ones_r)
    agg0 = agg(x, src, dst)
    h1, inv = _dense0(agg0, cnt2, x, W0l, b0.reshape(1, D), W0r,
                      g0.reshape(1, D), be0.reshape(1, D))
    agg1 = agg(h1, src, dst)
    h2 = _dense_relu(agg1, inv, h1, W1l, b1.reshape(1, D), W1r,
                     g1.reshape(1, D), be1.reshape(1, D))
    agg2 = agg(h2, src, dst)
    h3 = _dense_final(agg2, inv, h2, W2l, b2.reshape(1, D), W2r,
                      g2.reshape(1, D), be2.reshape(1, D))
    return h3
